# baseline (speedup 1.0000x reference)
import functools
import math

import jax
import jax.numpy as jnp
from jax import lax
from jax.experimental import pallas as pl
from jax.experimental.pallas import tpu as pltpu
from jax.experimental.pallas import tpu_sc as plsc

D_MODEL = 64
SCALE = math.sqrt(D_MODEL)
NC, NS = 2, 16
NW = NC * NS
SEQ = 50
NBATCH = 4096
VOCAB = 1000000
B_ROWS = NBATCH * SEQ
BPW = B_ROWS // NW
BB = NBATCH // NW
MACRO = 640
NMACRO = BPW // MACRO
NTC = (VOCAB + 127) // 128
TPW = (NTC + NW - 1) // NW
RBW = 128 * D_MODEL


@jax.jit
def _sc_embed(x_t, lut_t):
    mesh = plsc.VectorSubcoreMesh(core_axis_name="c", subcore_axis_name="s")

    @functools.partial(
        pl.kernel,
        out_type=jax.ShapeDtypeStruct((B_ROWS,), jnp.int32),
        mesh=mesh,
        scratch_types=[
            pltpu.VMEM((SEQ, BB), jnp.int32),
            pltpu.VMEM((BPW,), jnp.int32),
            pltpu.SemaphoreType.DMA,
        ],
        compiler_params=pltpu.CompilerParams(
            use_tc_tiling_on_sc=True, needs_layout_passes=False
        ),
    )
    def xbody(xt_hbm, idx_hbm, stage, idx_v, sem):
        wid = lax.axis_index("s") * NC + lax.axis_index("c")
        for s in range(SEQ):
            pltpu.async_copy(
                xt_hbm.at[s, pl.ds(wid * BB, BB)], stage.at[s], sem
            )
        for s in range(SEQ):
            pltpu.make_async_copy(
                xt_hbm.at[s, pl.ds(wid * BB, BB)], stage.at[s], sem
            ).wait()

        lane_dst = lax.iota(jnp.int32, 16) * SEQ

        def t_body(s, c):
            for u in range(BB // 16):
                b0 = u * 16
                vals = stage[s, pl.ds(b0, 16)]
                dst = lane_dst + (b0 * SEQ + s)
                plsc.store_scatter(idx_v, [dst], vals)
            return c

        lax.fori_loop(0, SEQ, t_body, 0)
        pltpu.sync_copy(idx_v, idx_hbm.at[pl.ds(wid * BPW, BPW)])

    @functools.partial(
        pl.kernel,
        out_type=jax.ShapeDtypeStruct((VOCAB * D_MODEL,), jnp.float32),
        mesh=mesh,
        scratch_types=[
            pltpu.VMEM((8, 8, 128), jnp.float32),
            pltpu.VMEM((8, 8, 128), jnp.float32),
            pltpu.VMEM((RBW,), jnp.float32),
            pltpu.VMEM((RBW,), jnp.float32),
            pltpu.SemaphoreType.DMA,
            pltpu.SemaphoreType.DMA,
            pltpu.SemaphoreType.DMA,
            pltpu.SemaphoreType.DMA,
        ],
        compiler_params=pltpu.CompilerParams(
            use_tc_tiling_on_sc=True, needs_layout_passes=False
        ),
    )
    def rbody(lt_hbm, rm_hbm, vb0, vb1, rb0, rb1, gr0, gr1, sr0, sr1):
        wid = lax.axis_index("s") * NC + lax.axis_index("c")
        base = wid * TPW
        lanes = lax.iota(jnp.int32, 16)
        i0 = lax.shift_right_logical(lanes, 3)
        i1 = lanes & 7
        zeros = lanes * 0

        def fire_read(rb, vb, sem, last):
            c0 = pl.multiple_of(rb * 128, 128)

            @pl.when(jnp.logical_not(last))
            def _():
                for g in range(8):
                    pltpu.async_copy(
                        lt_hbm.at[pl.ds(g * 8, 8), pl.ds(c0, 128)],
                        vb.at[g],
                        sem,
                    )

            @pl.when(last)
            def _():
                for g in range(8):
                    for s in range(8):
                        pltpu.async_copy(
                            lt_hbm.at[g * 8 + s, pl.ds(VOCAB - 64, 64)],
                            vb.at[g, s, pl.ds(0, 64)],
                            sem,
                        )

        def drain_read(rb, vb, sem, last):
            c0 = pl.multiple_of(rb * 128, 128)

            @pl.when(jnp.logical_not(last))
            def _():
                for g in range(8):
                    pltpu.make_async_copy(
                        lt_hbm.at[pl.ds(g * 8, 8), pl.ds(c0, 128)],
                        vb.at[g],
                        sem,
                    ).wait()

            @pl.when(last)
            def _():
                for g in range(8):
                    for s in range(8):
                        pltpu.make_async_copy(
                            lt_hbm.at[g * 8 + s, pl.ds(VOCAB - 64, 64)],
                            vb.at[g, s, pl.ds(0, 64)],
                            sem,
                        ).wait()

        lane_off = lanes * D_MODEL

        def transpose(vb, rb_buf, nrows, jshift):
            @plsc.parallel_loop(0, nrows // 16, unroll=2)
            def _(jg):
                j0 = jg * 16
                for g in range(8):
                    for s in range(8):
                        c = g * 8 + s
                        vals = vb[g, s, pl.ds(j0 + jshift, 16)]
                        dst = lane_off + (j0 * D_MODEL + c)
                        plsc.store_scatter(rb_buf, [dst], vals * SCALE)

        def fire_write(rb, rb_buf, sem, partial):
            @pl.when(jnp.logical_not(partial))
            def _():
                pltpu.async_copy(
                    rb_buf, rm_hbm.at[pl.ds(rb * RBW, RBW)], sem
                )

            @pl.when(partial)
            def _():
                pltpu.async_copy(
                    rb_buf.at[pl.ds(0, RBW // 2)],
                    rm_hbm.at[pl.ds((VOCAB - 64) * D_MODEL, RBW // 2)],
                    sem,
                )

        def drain_write(rb, rb_buf, sem, partial):
            @pl.when(jnp.logical_not(partial))
            def _():
                pltpu.make_async_copy(
                    rb_buf, rm_hbm.at[pl.ds(rb * RBW, RBW)], sem
                ).wait()

            @pl.when(partial)
            def _():
                pltpu.make_async_copy(
                    rb_buf.at[pl.ds(0, RBW // 2)],
                    rm_hbm.at[pl.ds((VOCAB - 64) * D_MODEL, RBW // 2)],
                    sem,
                ).wait()

        def prefetch(k, vb, sem):
            rb = base + k
            live = jnp.logical_and(k < TPW, rb < NTC)

            @pl.when(live)
            def _():
                fire_read(rb, vb, sem, rb == NTC - 1)

        def process(k, vb, rb_buf, gsem, ssem):
            rb = base + k
            live = jnp.logical_and(k < TPW, rb < NTC)

            @pl.when(live)
            def _():
                last = rb == NTC - 1
                drain_read(rb, vb, gsem, last)

                @pl.when(k >= 2)
                def _():
                    drain_write(rb - 2, rb_buf, ssem, rb - 2 == NTC - 1)

                @pl.when(jnp.logical_not(last))
                def _():
                    transpose(vb, rb_buf, 128, 0)

                @pl.when(last)
                def _():
                    transpose(vb, rb_buf, 64, 0)

                fire_write(rb, rb_buf, ssem, last)

        prefetch(0, vb0, gr0)
        prefetch(1, vb1, gr1)

        def pair_body(p, carry):
            k0 = 2 * p
            process(k0, vb0, rb0, gr0, sr0)
            prefetch(k0 + 2, vb0, gr0)
            process(k0 + 1, vb1, rb1, gr1, sr1)
            prefetch(k0 + 3, vb1, gr1)
            return carry

        lax.fori_loop(0, (TPW + 1) // 2, pair_body, 0)

        nlive = jnp.minimum(TPW, NTC - base)

        def tail_drain(parity, rb_buf, ssem):
            k = nlive - 1
            k = jnp.where((k % 2) == parity, k, k - 1)
            rb = base + k

            @pl.when(k >= 0)
            def _():
                drain_write(rb, rb_buf, ssem, rb == NTC - 1)

        tail_drain(0, rb0, sr0)
        tail_drain(1, rb1, sr1)

    @functools.partial(
        pl.kernel,
        out_type=jax.ShapeDtypeStruct((B_ROWS, D_MODEL), jnp.float32),
        mesh=mesh,
        scratch_types=[
            pltpu.VMEM((BPW,), jnp.int32),
            pltpu.VMEM((MACRO, D_MODEL), jnp.float32),
            pltpu.VMEM((MACRO, D_MODEL), jnp.float32),
            pltpu.SemaphoreType.DMA,
            pltpu.SemaphoreType.DMA,
            pltpu.SemaphoreType.DMA,
            pltpu.SemaphoreType.DMA,
        ],
        compiler_params=pltpu.CompilerParams(use_tc_tiling_on_sc=False),
    )
    def body(xn_hbm, lut_hbm, out_hbm, idx_v, buf0, buf1, g0, g1, s0, s1):
        wid = lax.axis_index("s") * NC + lax.axis_index("c")
        pltpu.sync_copy(xn_hbm.at[pl.ds(wid * BPW, BPW)], idx_v)

        GRP = 16
        UNROLL = 8
        NGRP = MACRO // GRP

        def fire_gathers(m, buf, sem):
            def g_body(g, c):
                for u in range(UNROLL):
                    off = g * (GRP * UNROLL) + u * GRP
                    iv = idx_v[pl.ds(m * MACRO + off, GRP)]
                    pltpu.async_copy(
                        lut_hbm.at[iv], buf.at[pl.ds(off, GRP)], sem
                    )
                return c

            lax.fori_loop(0, NGRP // UNROLL, g_body, 0)

        def drain_gathers(m, buf, sem):
            def g_body(g, c):
                for u in range(UNROLL):
                    off = g * (GRP * UNROLL) + u * GRP
                    iv = idx_v[pl.ds(m * MACRO + off, GRP)]
                    pltpu.make_async_copy(
                        lut_hbm.at[iv], buf.at[pl.ds(off, GRP)], sem
                    ).wait()
                return c

            lax.fori_loop(0, NGRP // UNROLL, g_body, 0)

        def fire_store(m, buf, sem):
            pltpu.async_copy(
                buf, out_hbm.at[pl.ds(wid * BPW + m * MACRO, MACRO)], sem
            )

        def drain_store(m, buf, sem):
            pltpu.make_async_copy(
                buf, out_hbm.at[pl.ds(wid * BPW + m * MACRO, MACRO)], sem
            ).wait()

        fire_gathers(0, buf0, g0)

        def pair_body(p, carry):
            m0 = 2 * p
            m1 = m0 + 1
            drain_gathers(m0, buf0, g0)

            @pl.when(p > 0)
            def _():
                drain_store(m1 - 2, buf1, s1)

            fire_gathers(m1, buf1, g1)
            fire_store(m0, buf0, s0)
            drain_gathers(m1, buf1, g1)
            drain_store(m0, buf0, s0)

            @pl.when(p < NMACRO // 2 - 1)
            def _():
                fire_gathers(m0 + 2, buf0, g0)

            fire_store(m1, buf1, s1)
            return carry

        lax.fori_loop(0, NMACRO // 2, pair_body, 0)
        drain_store(NMACRO - 1, buf1, s1)

    x_n = xbody(x_t)
    rm = rbody(lut_t).reshape(VOCAB, D_MODEL)
    return body(x_n, rm)


def kernel(x, lut):
    x_t = x.astype(jnp.int32).T
    out = _sc_embed(x_t, lut.T)
    return out.reshape(NBATCH, SEQ, D_MODEL)


# speedup vs baseline: 1.2986x; 1.2986x over previous
import functools
import math

import jax
import jax.numpy as jnp
from jax import lax
from jax.experimental import pallas as pl
from jax.experimental.pallas import tpu as pltpu
from jax.experimental.pallas import tpu_sc as plsc

D_MODEL = 64
SCALE = math.sqrt(D_MODEL)
NC, NS = 2, 16
NW = NC * NS
SEQ = 50
NBATCH = 4096
B_ROWS = NBATCH * SEQ
VOCAB = 1000000
BPW = B_ROWS // NW
BB = NBATCH // NW
MACRO = 640
NMACRO = BPW // MACRO


@jax.jit
def _sc_embed(x_t, lut_t):
    mesh = plsc.VectorSubcoreMesh(core_axis_name="c", subcore_axis_name="s")

    @functools.partial(
        pl.kernel,
        out_type=jax.ShapeDtypeStruct((B_ROWS,), jnp.int32),
        mesh=mesh,
        scratch_types=[
            pltpu.VMEM((SEQ, BB), jnp.int32),
            pltpu.VMEM((BPW,), jnp.int32),
            pltpu.SemaphoreType.DMA,
        ],
        compiler_params=pltpu.CompilerParams(
            use_tc_tiling_on_sc=True, needs_layout_passes=False
        ),
    )
    def xbody(xt_hbm, idx_hbm, stage, idx_v, sem):
        wid = lax.axis_index("s") * NC + lax.axis_index("c")
        for s in range(SEQ):
            pltpu.async_copy(
                xt_hbm.at[s, pl.ds(wid * BB, BB)], stage.at[s], sem
            )
        for s in range(SEQ):
            pltpu.make_async_copy(
                xt_hbm.at[s, pl.ds(wid * BB, BB)], stage.at[s], sem
            ).wait()

        lane_dst = lax.iota(jnp.int32, 16) * SEQ

        def t_body(s, c):
            for u in range(BB // 16):
                b0 = u * 16
                vals = stage[s, pl.ds(b0, 16)]
                dst = lane_dst + (b0 * SEQ + s)
                plsc.store_scatter(idx_v, [dst], vals)
            return c

        lax.fori_loop(0, SEQ, t_body, 0)
        pltpu.sync_copy(idx_v, idx_hbm.at[pl.ds(wid * BPW, BPW)])

    NTC = (VOCAB + 127) // 128
    TPW = (NTC + NW - 1) // NW
    RBW = 128 * D_MODEL

    @functools.partial(
        pl.kernel,
        out_type=jax.ShapeDtypeStruct((VOCAB * D_MODEL,), jnp.float32),
        mesh=mesh,
        scratch_types=[
            pltpu.VMEM((8, 8, 128), jnp.float32),
            pltpu.VMEM((8, 8, 128), jnp.float32),
            pltpu.VMEM((RBW,), jnp.float32),
            pltpu.VMEM((RBW,), jnp.float32),
            pltpu.SemaphoreType.DMA,
            pltpu.SemaphoreType.DMA,
            pltpu.SemaphoreType.DMA,
            pltpu.SemaphoreType.DMA,
        ],
        compiler_params=pltpu.CompilerParams(
            use_tc_tiling_on_sc=True, needs_layout_passes=False
        ),
    )
    def rbody(lt_hbm, rm_hbm, vb0, vb1, rb0, rb1, gr0, gr1, sr0, sr1):
        wid = lax.axis_index("s") * NC + lax.axis_index("c")
        base = wid * TPW
        lanes = lax.iota(jnp.int32, 16)
        i0 = lax.shift_right_logical(lanes, 3)
        i1 = lanes & 7
        zeros = lanes * 0

        def fire_read(rb, vb, sem, last):
            c0 = pl.multiple_of(rb * 128, 128)

            @pl.when(jnp.logical_not(last))
            def _():
                for g in range(8):
                    pltpu.async_copy(
                        lt_hbm.at[pl.ds(g * 8, 8), pl.ds(c0, 128)],
                        vb.at[g],
                        sem,
                    )

            @pl.when(last)
            def _():
                for g in range(8):
                    for s in range(8):
                        pltpu.async_copy(
                            lt_hbm.at[g * 8 + s, pl.ds(VOCAB - 64, 64)],
                            vb.at[g, s, pl.ds(0, 64)],
                            sem,
                        )

        def drain_read(rb, vb, sem, last):
            c0 = pl.multiple_of(rb * 128, 128)

            @pl.when(jnp.logical_not(last))
            def _():
                for g in range(8):
                    pltpu.make_async_copy(
                        lt_hbm.at[pl.ds(g * 8, 8), pl.ds(c0, 128)],
                        vb.at[g],
                        sem,
                    ).wait()

            @pl.when(last)
            def _():
                for g in range(8):
                    for s in range(8):
                        pltpu.make_async_copy(
                            lt_hbm.at[g * 8 + s, pl.ds(VOCAB - 64, 64)],
                            vb.at[g, s, pl.ds(0, 64)],
                            sem,
                        ).wait()

        def transpose(vb, rb_buf, nrows):
            @plsc.parallel_loop(0, nrows, unroll=8)
            def _(j):
                jj = zeros + j
                for u in range(4):
                    vals = plsc.load_gather(vb, [i0 + 2 * u, i1, jj])
                    rb_buf[pl.ds(j * D_MODEL + u * 16, 16)] = vals * SCALE

        def fire_write(rb, rb_buf, sem, partial):
            @pl.when(jnp.logical_not(partial))
            def _():
                pltpu.async_copy(rb_buf, rm_hbm.at[pl.ds(rb * RBW, RBW)], sem)

            @pl.when(partial)
            def _():
                pltpu.async_copy(
                    rb_buf.at[pl.ds(0, RBW // 2)],
                    rm_hbm.at[pl.ds((VOCAB - 64) * D_MODEL, RBW // 2)],
                    sem,
                )

        def drain_write(rb, rb_buf, sem, partial):
            @pl.when(jnp.logical_not(partial))
            def _():
                pltpu.make_async_copy(
                    rb_buf, rm_hbm.at[pl.ds(rb * RBW, RBW)], sem
                ).wait()

            @pl.when(partial)
            def _():
                pltpu.make_async_copy(
                    rb_buf.at[pl.ds(0, RBW // 2)],
                    rm_hbm.at[pl.ds((VOCAB - 64) * D_MODEL, RBW // 2)],
                    sem,
                ).wait()

        def prefetch(k, vb, sem):
            rb = base + k
            live = jnp.logical_and(k < TPW, rb < NTC)

            @pl.when(live)
            def _():
                fire_read(rb, vb, sem, rb == NTC - 1)

        def process(k, vb, rb_buf, gsem, ssem):
            rb = base + k
            live = jnp.logical_and(k < TPW, rb < NTC)

            @pl.when(live)
            def _():
                last = rb == NTC - 1
                drain_read(rb, vb, gsem, last)

                @pl.when(k >= 2)
                def _():
                    drain_write(rb - 2, rb_buf, ssem, rb - 2 == NTC - 1)

                @pl.when(jnp.logical_not(last))
                def _():
                    transpose(vb, rb_buf, 128)

                @pl.when(last)
                def _():
                    transpose(vb, rb_buf, 64)

                fire_write(rb, rb_buf, ssem, last)

        prefetch(0, vb0, gr0)
        prefetch(1, vb1, gr1)

        def r_pair(p, carry):
            k0 = 2 * p
            process(k0, vb0, rb0, gr0, sr0)
            prefetch(k0 + 2, vb0, gr0)
            process(k0 + 1, vb1, rb1, gr1, sr1)
            prefetch(k0 + 3, vb1, gr1)
            return carry

        lax.fori_loop(0, (TPW + 1) // 2, r_pair, 0)

        nlive = jnp.minimum(TPW, NTC - base)

        def tail_drain(parity, rb_buf, ssem):
            k = nlive - 1
            k = jnp.where((k % 2) == parity, k, k - 1)
            rb = base + k

            @pl.when(k >= 0)
            def _():
                drain_write(rb, rb_buf, ssem, rb == NTC - 1)

        tail_drain(0, rb0, sr0)
        tail_drain(1, rb1, sr1)

    @functools.partial(
        pl.kernel,
        out_type=jax.ShapeDtypeStruct((B_ROWS, D_MODEL), jnp.float32),
        mesh=mesh,
        scratch_types=[
            pltpu.VMEM((BPW,), jnp.int32),
            pltpu.VMEM((MACRO, D_MODEL), jnp.float32),
            pltpu.VMEM((MACRO, D_MODEL), jnp.float32),
            pltpu.SemaphoreType.DMA,
            pltpu.SemaphoreType.DMA,
            pltpu.SemaphoreType.DMA,
            pltpu.SemaphoreType.DMA,
        ],
        compiler_params=pltpu.CompilerParams(use_tc_tiling_on_sc=False),
    )
    def body(xn_hbm, lut_hbm, out_hbm, idx_v, buf0, buf1, g0, g1, s0, s1):
        wid = lax.axis_index("s") * NC + lax.axis_index("c")
        pltpu.sync_copy(xn_hbm.at[pl.ds(wid * BPW, BPW)], idx_v)

        GRP = 16
        UNROLL = 8
        NGRP = MACRO // GRP

        def fire_gathers(m, buf, sem):
            def g_body(g, c):
                for u in range(UNROLL):
                    off = g * (GRP * UNROLL) + u * GRP
                    iv = idx_v[pl.ds(m * MACRO + off, GRP)]
                    pltpu.async_copy(
                        lut_hbm.at[iv], buf.at[pl.ds(off, GRP)], sem
                    )
                return c

            lax.fori_loop(0, NGRP // UNROLL, g_body, 0)

        def drain_gathers(m, buf, sem):
            def g_body(g, c):
                for u in range(UNROLL):
                    off = g * (GRP * UNROLL) + u * GRP
                    iv = idx_v[pl.ds(m * MACRO + off, GRP)]
                    pltpu.make_async_copy(
                        lut_hbm.at[iv], buf.at[pl.ds(off, GRP)], sem
                    ).wait()
                return c

            lax.fori_loop(0, NGRP // UNROLL, g_body, 0)

        def fire_store(m, buf, sem):
            pltpu.async_copy(
                buf, out_hbm.at[pl.ds(wid * BPW + m * MACRO, MACRO)], sem
            )

        def drain_store(m, buf, sem):
            pltpu.make_async_copy(
                buf, out_hbm.at[pl.ds(wid * BPW + m * MACRO, MACRO)], sem
            ).wait()

        def scale(buf):
            @plsc.parallel_loop(0, MACRO, unroll=4)
            def _(r):
                for c in range(D_MODEL // 16):
                    sl = pl.ds(c * 16, 16)
                    buf[r, sl] = buf[r, sl] * SCALE

        fire_gathers(0, buf0, g0)

        def pair_body(p, carry):
            m0 = 2 * p
            m1 = m0 + 1
            drain_gathers(m0, buf0, g0)

            @pl.when(p > 0)
            def _():
                drain_store(m1 - 2, buf1, s1)

            fire_gathers(m1, buf1, g1)
            fire_store(m0, buf0, s0)
            drain_gathers(m1, buf1, g1)
            drain_store(m0, buf0, s0)

            @pl.when(p < NMACRO // 2 - 1)
            def _():
                fire_gathers(m0 + 2, buf0, g0)

            fire_store(m1, buf1, s1)
            return carry

        lax.fori_loop(0, NMACRO // 2, pair_body, 0)
        drain_store(NMACRO - 1, buf1, s1)

    x_n = xbody(x_t)
    rm = rbody(lut_t).reshape(VOCAB, D_MODEL)
    return body(x_n, rm)


def kernel(x, lut):
    x_t = x.astype(jnp.int32).T
    out = _sc_embed(x_t, lut.T)
    return out.reshape(NBATCH, SEQ, D_MODEL)
